# baseline (speedup 1.0000x reference)
import numpy as np
import jax
import jax.numpy as jnp
from jax.experimental import pallas as pl
from jax.experimental.pallas import tpu as pltpu

_N = 24
_NCODONS = 2
_NROWS = 65536


def _build_tables(n=_NCODONS):
    offset = 8 + 3 * n
    idx = [[0, 0], [0, 1], [1, 2], [2, 3]]
    idx += [[3 + i * 3, 4 + i * 3] for i in range(n)]
    idx += [[4 + i * 3, 5 + i * 3] for i in range(n)]
    idx += [[5 + i * 3, 6 + i * 3] for i in range(n)]
    idx += [[3 + i * 3, offset + i * 3] for i in range(n + 1)]
    idx += [[3 + n * 3, 4 + n * 3]]
    idx += [[offset + i * 3, offset + 1 + i * 3] for i in range(n + 1)]
    idx += [[offset + 1 + i * 3, offset + 2 + i * 3] for i in range(n + 1)]
    idx += [[offset + 2 + i * 3, 4 + i * 3] for i in range(n + 1)]
    idx += [[offset + 2 + i * 3, offset + i * 3] for i in range(n + 1)]
    i_del = [3 + i * 3 for i in range(n) for j in range(n - i)]
    j_del = [4 + j * 3 for i in range(1, n + 1) for j in range(i, n + 1)]
    idx += [[i, j] for i, j in zip(i_del, j_del)]
    idx += [[4 + n * 3, 5 + n * 3]]
    idx += [[5 + n * 3, 6 + n * 3]]
    idx += [[6 + n * 3, 7 + n * 3]]
    idx += [[7 + n * 3, 7 + n * 3]]
    idx += [[7 + n * 3, 8 + n * 3 + (n + 1) * 3]]
    idx += [[8 + n * 3 + (n + 1) * 3, 8 + n * 3 + (n + 1) * 3]]
    idx = np.array(idx, np.int32)

    sym = []
    sym += [(1.0, -1.0, (0,)), (0.0, 1.0, (0,))]
    sym += [(1.0, 0.0, ())] * 2
    sym += [(0.0, 1.0, (1 + i,)) for i in range(n)]
    sym += [(1.0, 0.0, ())] * n
    sym += [(1.0, 0.0, ())] * n
    k = 1 + n
    sym += [(0.0, 1.0, (k + i,)) for i in range(n + 1)]
    sym += [(1.0, -1.0, (k + n,))]
    k += n + 1
    sym += [(1.0, 0.0, ())] * (n + 1)
    sym += [(1.0, 0.0, ())] * (n + 1)
    sym += [(0.0, 1.0, (k + i,)) for i in range(n + 1)]
    sym += [(1.0, -1.0, (k + i,)) for i in range(n + 1)]
    k += n + 1
    exps = [int((j - i) / 3) for i, j in zip(i_del, j_del)]
    sym += [(1.0, -1.0, (k,) * (e + 1)) for e in exps]
    sym += [(1.0, 0.0, ())] * 6
    assert len(sym) == len(idx)

    ne = len(sym)
    npad = ((ne + 15) // 16) * 16
    c0 = np.ones(npad, np.float32)
    c1 = np.zeros(npad, np.float32)
    g = np.full((npad, 3), 10, np.int32)
    rows = np.zeros(npad, np.int32)
    cols = np.zeros(npad, np.int32)
    for e, (a, b, gt) in enumerate(sym):
        c0[e], c1[e] = a, b
        for j, gi in enumerate(gt):
            g[e, j] = gi
        rows[e], cols[e] = idx[e]
    return ne, npad, c0, c1, g, rows, cols


_NE, _NP, _C0, _C1, _G, _ROWS, _COLS = _build_tables()

_GH = np.zeros((3 * 16, _NP), np.float32)
for _e in range(_NP):
    for _j in range(3):
        _GH[_j * 16 + _G[_e, _j], _e] = 1.0
_QROW = np.zeros((_N, _NP), np.float32)
_PCOL = np.zeros((_NP, _N), np.float32)
for _e in range(_NE):
    _QROW[_ROWS[_e], _e] = 1.0
    _PCOL[_e, _COLS[_e]] = 1.0
_CO = np.zeros((4, _NP), np.float32)
_CO[0] = _C0
_CO[1] = _C1
_CO[2, :_NE] = 1.0
_CO[3] = 1.0 - _CO[2]
_TBL = np.concatenate([_GH, _CO, _QROW, _PCOL.T], axis=0)

_BLK = 65536


def _fused_body(w_ref, t_ref, a_ref, o_ref, t_scr):
    @pl.when(pl.program_id(0) == 0)
    def _():
        w = w_ref[...]
        wa = jnp.dot(w, t_ref[0:16, :])
        wb = jnp.dot(w, t_ref[16:32, :])
        wc = jnp.dot(w, t_ref[32:48, :])
        val = t_ref[48:49, :] + t_ref[49:50, :] * wa * wb * wc
        e = jnp.exp(val) * t_ref[50:51, :]
        q = t_ref[52:76, :]
        rs = jnp.dot(e, q.T)
        denom = jnp.dot(rs, q) + t_ref[51:52, :]
        a = e / denom
        from jax import lax as _lax
        amat = _lax.dot_general(
            q * a, t_ref[76:100, :], (((1,), (1,)), ((), ())))
        t_scr[...] = amat.astype(jnp.bfloat16)
    o_ref[...] = jnp.dot(a_ref[...], t_scr[...],
                         preferred_element_type=jnp.float32).astype(jnp.bfloat16)


@jax.jit
def kernel(alpha, transition_kernel):
    w = jnp.concatenate([transition_kernel.astype(jnp.float32),
                         jnp.ones((6,), jnp.float32)]).reshape(1, 16)
    a16 = alpha.astype(jnp.bfloat16)
    nblk = _NROWS // _BLK
    zmap = lambda i: (0, 0)
    out = pl.pallas_call(
        _fused_body,
        grid=(nblk,),
        in_specs=[
            pl.BlockSpec((1, 16), zmap),
            pl.BlockSpec((100, _NP), zmap),
            pl.BlockSpec((_BLK, _N), lambda i: (i, 0)),
        ],
        out_specs=pl.BlockSpec((_BLK, _N), lambda i: (i, 0)),
        out_shape=jax.ShapeDtypeStruct((_NROWS, _N), jnp.bfloat16),
        scratch_shapes=[pltpu.VMEM((_N, _N), jnp.bfloat16)],
    )(w, jnp.asarray(_TBL), a16)
    return out.astype(jnp.float32)


# speedup vs baseline: 1.1006x; 1.1006x over previous
import numpy as np
import jax
import jax.numpy as jnp
from jax.experimental import pallas as pl
from jax.experimental.pallas import tpu as pltpu

_N = 24
_NCODONS = 2
_NROWS = 65536


def _build_tables(n=_NCODONS):
    offset = 8 + 3 * n
    idx = [[0, 0], [0, 1], [1, 2], [2, 3]]
    idx += [[3 + i * 3, 4 + i * 3] for i in range(n)]
    idx += [[4 + i * 3, 5 + i * 3] for i in range(n)]
    idx += [[5 + i * 3, 6 + i * 3] for i in range(n)]
    idx += [[3 + i * 3, offset + i * 3] for i in range(n + 1)]
    idx += [[3 + n * 3, 4 + n * 3]]
    idx += [[offset + i * 3, offset + 1 + i * 3] for i in range(n + 1)]
    idx += [[offset + 1 + i * 3, offset + 2 + i * 3] for i in range(n + 1)]
    idx += [[offset + 2 + i * 3, 4 + i * 3] for i in range(n + 1)]
    idx += [[offset + 2 + i * 3, offset + i * 3] for i in range(n + 1)]
    i_del = [3 + i * 3 for i in range(n) for j in range(n - i)]
    j_del = [4 + j * 3 for i in range(1, n + 1) for j in range(i, n + 1)]
    idx += [[i, j] for i, j in zip(i_del, j_del)]
    idx += [[4 + n * 3, 5 + n * 3]]
    idx += [[5 + n * 3, 6 + n * 3]]
    idx += [[6 + n * 3, 7 + n * 3]]
    idx += [[7 + n * 3, 7 + n * 3]]
    idx += [[7 + n * 3, 8 + n * 3 + (n + 1) * 3]]
    idx += [[8 + n * 3 + (n + 1) * 3, 8 + n * 3 + (n + 1) * 3]]
    idx = np.array(idx, np.int32)

    sym = []
    sym += [(1.0, -1.0, (0,)), (0.0, 1.0, (0,))]
    sym += [(1.0, 0.0, ())] * 2
    sym += [(0.0, 1.0, (1 + i,)) for i in range(n)]
    sym += [(1.0, 0.0, ())] * n
    sym += [(1.0, 0.0, ())] * n
    k = 1 + n
    sym += [(0.0, 1.0, (k + i,)) for i in range(n + 1)]
    sym += [(1.0, -1.0, (k + n,))]
    k += n + 1
    sym += [(1.0, 0.0, ())] * (n + 1)
    sym += [(1.0, 0.0, ())] * (n + 1)
    sym += [(0.0, 1.0, (k + i,)) for i in range(n + 1)]
    sym += [(1.0, -1.0, (k + i,)) for i in range(n + 1)]
    k += n + 1
    exps = [int((j - i) / 3) for i, j in zip(i_del, j_del)]
    sym += [(1.0, -1.0, (k,) * (e + 1)) for e in exps]
    sym += [(1.0, 0.0, ())] * 6
    assert len(sym) == len(idx)

    ne = len(sym)
    npad = ((ne + 15) // 16) * 16
    c0 = np.ones(npad, np.float32)
    c1 = np.zeros(npad, np.float32)
    g = np.full((npad, 3), 10, np.int32)
    rows = np.zeros(npad, np.int32)
    cols = np.zeros(npad, np.int32)
    for e, (a, b, gt) in enumerate(sym):
        c0[e], c1[e] = a, b
        for j, gi in enumerate(gt):
            g[e, j] = gi
        rows[e], cols[e] = idx[e]
    return ne, npad, c0, c1, g, rows, cols


_NE, _NP, _C0, _C1, _G, _ROWS, _COLS = _build_tables()

_GH = np.zeros((3 * 16, _NP), np.float32)
for _e in range(_NP):
    for _j in range(3):
        _GH[_j * 16 + _G[_e, _j], _e] = 1.0
_QROW = np.zeros((_N, _NP), np.float32)
_PCOL = np.zeros((_NP, _N), np.float32)
for _e in range(_NE):
    _QROW[_ROWS[_e], _e] = 1.0
    _PCOL[_e, _COLS[_e]] = 1.0
_CO = np.zeros((4, _NP), np.float32)
_CO[0] = _C0
_CO[1] = _C1
_CO[2, :_NE] = 1.0
_CO[3] = 1.0 - _CO[2]
_TBL = np.concatenate([_GH, _CO, _QROW, _PCOL.T], axis=0)

_BLK = 32768


def _fused_body(w_ref, t_ref, a_ref, o_ref, t_scr):
    @pl.when(pl.program_id(0) == 0)
    def _():
        w = w_ref[...]
        wa = jnp.dot(w, t_ref[0:16, :])
        wb = jnp.dot(w, t_ref[16:32, :])
        wc = jnp.dot(w, t_ref[32:48, :])
        val = t_ref[48:49, :] + t_ref[49:50, :] * wa * wb * wc
        e = jnp.exp(val) * t_ref[50:51, :]
        q = t_ref[52:76, :]
        rs = jnp.dot(e, q.T)
        denom = jnp.dot(rs, q) + t_ref[51:52, :]
        a = e / denom
        from jax import lax as _lax
        amat = _lax.dot_general(
            q * a, t_ref[76:100, :], (((1,), (1,)), ((), ())))
        t_scr[...] = amat.astype(jnp.bfloat16)
    o_ref[...] = jnp.dot(a_ref[...], t_scr[...],
                         preferred_element_type=jnp.float32).astype(jnp.bfloat16)


@jax.jit
def kernel(alpha, transition_kernel):
    w = jnp.concatenate([transition_kernel.astype(jnp.float32),
                         jnp.ones((6,), jnp.float32)]).reshape(1, 16)
    a16 = alpha.astype(jnp.bfloat16)
    nblk = _NROWS // _BLK
    zmap = lambda i: (0, 0)
    out = pl.pallas_call(
        _fused_body,
        grid=(nblk,),
        in_specs=[
            pl.BlockSpec((1, 16), zmap),
            pl.BlockSpec((100, _NP), zmap),
            pl.BlockSpec((_BLK, _N), lambda i: (i, 0)),
        ],
        out_specs=pl.BlockSpec((_BLK, _N), lambda i: (i, 0)),
        out_shape=jax.ShapeDtypeStruct((_NROWS, _N), jnp.bfloat16),
        scratch_shapes=[pltpu.VMEM((_N, _N), jnp.bfloat16)],
    )(w, jnp.asarray(_TBL), a16)
    return out.astype(jnp.float32)
